# Initial kernel scaffold; baseline (speedup 1.0000x reference)
import jax
import jax.numpy as jnp
from jax.experimental import pallas as pl


def kernel(context_ids, center_ids, labels, context_table, center_table):
    raise NotImplementedError("write your pallas kernel here")


# speedup vs baseline: 6.9355x; 6.9355x over previous
import functools

import jax
import jax.numpy as jnp
from jax import lax
from jax.experimental import pallas as pl
from jax.experimental.pallas import tpu as pltpu
from jax.experimental.pallas import tpu_sc as plsc

NC = 2
NS = 16
NW = NC * NS
LANES = 16

B = 16384
L = 20
D = 64
DC = D // LANES
CHUNK = B // NW
SB = 32
NSB = CHUNK // SB
IDX_W = 128
NG = (SB * L) // IDX_W


def _sc_scores_body(ctx_idx_hbm, cen_idx_hbm, ctx_tab, cen_tab, out_hbm,
                    idx_v, cidx_v, rows_v, crows_v, t0_v, partA_v, partB_v,
                    scores_v, sem):
  wid = lax.axis_index("s") * NC + lax.axis_index("c")

  pltpu.sync_copy(ctx_idx_hbm.at[pl.ds(wid * (CHUNK * L), CHUNK * L)], idx_v)
  pltpu.sync_copy(cen_idx_hbm.at[pl.ds(wid * CHUNK, CHUNK)], cidx_v)
  pltpu.sync_copy(ctx_tab.at[pl.ds(0, 8)], t0_v)

  t0c = [t0_v[0, pl.ds(c * LANES, LANES)] for c in range(DC)]
  lane = lax.iota(jnp.int32, LANES)

  def sub_block(i, carry):
    copies = []
    for g in range(NG):
      copies.append(pltpu.async_copy(
          ctx_tab.at[idx_v.at[pl.ds(i * (SB * L) + g * IDX_W, IDX_W)]],
          rows_v.at[pl.ds(g * IDX_W, IDX_W)], sem))
    copies.append(pltpu.async_copy(
        cen_tab.at[cidx_v.at[pl.ds(i * SB, SB)]], crows_v, sem))
    for cp in copies:
      cp.wait()

    def row_body(e, c2):
      base = e * L
      acc = [rows_v[base, pl.ds(c * LANES, LANES)] for c in range(DC)]
      for j in range(1, L):
        for c in range(DC):
          acc[c] = acc[c] + rows_v[base + j, pl.ds(c * LANES, LANES)]
      cen = [crows_v[e, pl.ds(c * LANES, LANES)] for c in range(DC)]
      pA = (acc[0] * cen[0] + acc[1] * cen[1]) + (acc[2] * cen[2] +
                                                  acc[3] * cen[3])
      pB = (t0c[0] * cen[0] + t0c[1] * cen[1]) + (t0c[2] * cen[2] +
                                                   t0c[3] * cen[3])
      partA_v[pl.ds(e * LANES, LANES)] = pA
      partB_v[pl.ds(e * LANES, LANES)] = pB
      return c2

    lax.fori_loop(0, SB, row_body, 0)

    for g in range(SB // LANES):
      n0 = jnp.zeros((LANES,), jnp.int32)
      idbase = (i * SB + g * LANES) * L
      for j in range(L):
        ids = plsc.load_gather(idx_v, [idbase + lane * L + j])
        n0 = n0 + jnp.where(ids == 0, 1, 0).astype(jnp.int32)
      pbase = g * (LANES * LANES) + lane * LANES
      sA = jnp.zeros((LANES,), jnp.float32)
      sB = jnp.zeros((LANES,), jnp.float32)
      for c in range(LANES):
        sA = sA + plsc.load_gather(partA_v, [pbase + c])
        sB = sB + plsc.load_gather(partB_v, [pbase + c])
      n0f = n0.astype(jnp.float32)
      score = (sA - n0f * sB) / (jnp.float32(L) - n0f)
      scores_v[pl.ds(i * SB + g * LANES, LANES)] = score
    return carry

  lax.fori_loop(0, NSB, sub_block, 0)
  pltpu.sync_copy(scores_v, out_hbm.at[pl.ds(wid * CHUNK, CHUNK)])


_sc_scores = functools.partial(
    pl.kernel,
    out_type=jax.ShapeDtypeStruct((B,), jnp.float32),
    mesh=plsc.VectorSubcoreMesh(core_axis_name="c", subcore_axis_name="s"),
    compiler_params=pltpu.CompilerParams(
        needs_layout_passes=False, use_tc_tiling_on_sc=False),
    scratch_types=[
        pltpu.VMEM((CHUNK * L,), jnp.int32),
        pltpu.VMEM((CHUNK,), jnp.int32),
        pltpu.VMEM((SB * L, D), jnp.float32),
        pltpu.VMEM((SB, D), jnp.float32),
        pltpu.VMEM((8, D), jnp.float32),
        pltpu.VMEM((SB * LANES,), jnp.float32),
        pltpu.VMEM((SB * LANES,), jnp.float32),
        pltpu.VMEM((CHUNK,), jnp.float32),
        pltpu.SemaphoreType.DMA,
    ],
)(_sc_scores_body)


def _tc_loss_body(scores_ref, labels_ref, out_ref):
  s = scores_ref[...]
  y = labels_ref[...]
  p = jax.nn.sigmoid(s)
  ll = -(y * jnp.log(p + 1e-08) + (1.0 - y) * jnp.log(1.0 - p + 1e-08))
  out_ref[0, 0] = jnp.sum(ll) * (1.0 / B)


def kernel(context_ids, center_ids, labels, context_table, center_table):
  ctx1d = context_ids.astype(jnp.int32).reshape(B * L)
  scores = _sc_scores(ctx1d, center_ids, context_table, center_table)
  loss = pl.pallas_call(
      _tc_loss_body,
      out_shape=jax.ShapeDtypeStruct((1, 1), jnp.float32),
      out_specs=pl.BlockSpec(memory_space=pltpu.SMEM),
  )(scores.reshape(128, 128), labels.reshape(128, 128))
  return loss[0, 0]


# speedup vs baseline: 8.0741x; 1.1642x over previous
import functools

import jax
import jax.numpy as jnp
from jax import lax
from jax.experimental import pallas as pl
from jax.experimental.pallas import tpu as pltpu
from jax.experimental.pallas import tpu_sc as plsc

NC = 2
NS = 16
NW = NC * NS
LANES = 16

B = 16384
L = 20
D = 64
DC = D // LANES
CHUNK = B // NW
SB = 32
NSB = CHUNK // SB
IDX_W = 128
NG = (SB * L) // IDX_W


def _sc_scores_body(ctx_idx_hbm, cen_idx_hbm, ctx_tab, cen_tab, out_hbm,
                    idx_v, cidx_v, rows0_v, rows1_v, crows0_v, crows1_v,
                    t0_v, partA_v, partB_v, scores_v, sem0, sem1):
  wid = lax.axis_index("s") * NC + lax.axis_index("c")

  pltpu.sync_copy(ctx_idx_hbm.at[pl.ds(wid * (CHUNK * L), CHUNK * L)], idx_v)
  pltpu.sync_copy(cen_idx_hbm.at[pl.ds(wid * CHUNK, CHUNK)], cidx_v)
  pltpu.sync_copy(ctx_tab.at[pl.ds(0, 8)], t0_v)

  t0c = [t0_v[0, pl.ds(c * LANES, LANES)] for c in range(DC)]
  lane = lax.iota(jnp.int32, LANES)

  def descr(i, rows_v, crows_v, sem):
    return (
        pltpu.make_async_copy(
            ctx_tab.at[idx_v.at[pl.ds(i * (SB * L), SB * L)]], rows_v, sem),
        pltpu.make_async_copy(
            cen_tab.at[cidx_v.at[pl.ds(i * SB, SB)]], crows_v, sem),
    )

  def issue(i, rows_v, crows_v, sem):
    for d in descr(i, rows_v, crows_v, sem):
      d.start()

  def wait(i, rows_v, crows_v, sem):
    for d in descr(i, rows_v, crows_v, sem):
      d.wait()

  def compute(i, rows_v, crows_v):
    def row_body(e, c2):
      base = e * L
      acc = [rows_v[base, pl.ds(c * LANES, LANES)] for c in range(DC)]
      for j in range(1, L):
        for c in range(DC):
          acc[c] = acc[c] + rows_v[base + j, pl.ds(c * LANES, LANES)]
      cen = [crows_v[e, pl.ds(c * LANES, LANES)] for c in range(DC)]
      pA = (acc[0] * cen[0] + acc[1] * cen[1]) + (acc[2] * cen[2] +
                                                  acc[3] * cen[3])
      pB = (t0c[0] * cen[0] + t0c[1] * cen[1]) + (t0c[2] * cen[2] +
                                                   t0c[3] * cen[3])
      partA_v[pl.ds(e * LANES, LANES)] = pA
      partB_v[pl.ds(e * LANES, LANES)] = pB
      return c2

    lax.fori_loop(0, SB, row_body, 0)

    for g in range(SB // LANES):
      n0 = jnp.zeros((LANES,), jnp.int32)
      idbase = (i * SB + g * LANES) * L
      for j in range(L):
        ids = plsc.load_gather(idx_v, [idbase + lane * L + j])
        n0 = n0 + jnp.where(ids == 0, 1, 0).astype(jnp.int32)
      pbase = g * (LANES * LANES) + lane * LANES
      sA = jnp.zeros((LANES,), jnp.float32)
      sB = jnp.zeros((LANES,), jnp.float32)
      for c in range(LANES):
        sA = sA + plsc.load_gather(partA_v, [pbase + c])
        sB = sB + plsc.load_gather(partB_v, [pbase + c])
      n0f = n0.astype(jnp.float32)
      score = (sA - n0f * sB) / (jnp.float32(L) - n0f)
      scores_v[pl.ds(i * SB + g * LANES, LANES)] = score

  issue(0, rows0_v, crows0_v, sem0)

  def sub_block(i, carry):
    def even():
      wait(i, rows0_v, crows0_v, sem0)

      @pl.when(i + 1 < NSB)
      def _():
        issue(i + 1, rows1_v, crows1_v, sem1)

      compute(i, rows0_v, crows0_v)

    def odd():
      wait(i, rows1_v, crows1_v, sem1)

      @pl.when(i + 1 < NSB)
      def _():
        issue(i + 1, rows0_v, crows0_v, sem0)

      compute(i, rows1_v, crows1_v)

    lax.cond(lax.rem(i, 2) == 0, even, odd)
    return carry

  lax.fori_loop(0, NSB, sub_block, 0)
  pltpu.sync_copy(scores_v, out_hbm.at[pl.ds(wid * CHUNK, CHUNK)])


_sc_scores = functools.partial(
    pl.kernel,
    out_type=jax.ShapeDtypeStruct((B,), jnp.float32),
    mesh=plsc.VectorSubcoreMesh(core_axis_name="c", subcore_axis_name="s"),
    compiler_params=pltpu.CompilerParams(
        needs_layout_passes=False, use_tc_tiling_on_sc=False),
    scratch_types=[
        pltpu.VMEM((CHUNK * L,), jnp.int32),
        pltpu.VMEM((CHUNK,), jnp.int32),
        pltpu.VMEM((SB * L, D), jnp.float32),
        pltpu.VMEM((SB * L, D), jnp.float32),
        pltpu.VMEM((SB, D), jnp.float32),
        pltpu.VMEM((SB, D), jnp.float32),
        pltpu.VMEM((8, D), jnp.float32),
        pltpu.VMEM((SB * LANES,), jnp.float32),
        pltpu.VMEM((SB * LANES,), jnp.float32),
        pltpu.VMEM((CHUNK,), jnp.float32),
        pltpu.SemaphoreType.DMA,
        pltpu.SemaphoreType.DMA,
    ],
)(_sc_scores_body)


def _tc_loss_body(scores_ref, labels_ref, out_ref):
  s = scores_ref[...]
  y = labels_ref[...]
  p = jax.nn.sigmoid(s)
  ll = -(y * jnp.log(p + 1e-08) + (1.0 - y) * jnp.log(1.0 - p + 1e-08))
  out_ref[0, 0] = jnp.sum(ll) * (1.0 / B)


def kernel(context_ids, center_ids, labels, context_table, center_table):
  ctx1d = context_ids.astype(jnp.int32).reshape(B * L)
  scores = _sc_scores(ctx1d, center_ids, context_table, center_table)
  loss = pl.pallas_call(
      _tc_loss_body,
      out_shape=jax.ShapeDtypeStruct((1, 1), jnp.float32),
      out_specs=pl.BlockSpec(memory_space=pltpu.SMEM),
  )(scores.reshape(128, 128), labels.reshape(128, 128))
  return loss[0, 0]
